# baseline (speedup 1.0000x reference)
import jax
import jax.numpy as jnp
from jax.experimental import pallas as pl
from jax.experimental.pallas import tpu as pltpu

_MEMORY_SIZE = 65536
_KEY_DIM = 64
_VALUE_DIM = 64
_BQ = 1024
_BM = 2048
_BMP = _BM // 2
_NUM_M_BLOCKS = _MEMORY_SIZE // _BM
_SCALE = 1.4426950408889634 / (_KEY_DIM ** 0.5)


def _attn_kernel(q_ref, k_ref, v_ref, wq_ref, bq_ref, wv_ref, bv_ref,
                 o_ref, q_scratch, qn_scratch, acc_scratch, m_scratch):
    i = pl.program_id(0)

    @pl.when(i == 0)
    def _init():
        q = jax.lax.dot_general(q_ref[...], wq_ref[...],
                                (((1,), (0,)), ((), ())),
                                preferred_element_type=jnp.float32)
        q_b = ((q + bq_ref[...]) * _SCALE).astype(jnp.bfloat16)
        q_scratch[...] = q_b
        q32 = q_b.astype(jnp.float32)
        qn = jnp.sqrt(jnp.sum(q32 * q32, axis=1, keepdims=True))
        qn_scratch[...] = jnp.broadcast_to(qn, qn_scratch.shape)
        m_scratch[...] = jnp.full_like(m_scratch, -jnp.inf)
        acc_scratch[...] = jnp.zeros_like(acc_scratch)

    kp = k_ref[...]
    ksq2 = kp * kp
    ksq_e = jnp.sum(ksq2[:, :_KEY_DIM], axis=1, keepdims=True)
    ksq_o = jnp.sum(ksq2[:, _KEY_DIM:], axis=1, keepdims=True)
    kmax = jnp.sqrt(jnp.max(jnp.maximum(ksq_e, ksq_o))) * 1.01
    kp_b = kp.astype(jnp.bfloat16)
    q_b = q_scratch[...]
    s_e = jax.lax.dot_general(q_b, kp_b[:, :_KEY_DIM],
                              (((1,), (1,)), ((), ())),
                              preferred_element_type=jnp.float32)
    s_o = jax.lax.dot_general(q_b, kp_b[:, _KEY_DIM:],
                              (((1,), (1,)), ((), ())),
                              preferred_element_type=jnp.float32)
    m_prev = m_scratch[...]
    m_next = jnp.maximum(m_prev, qn_scratch[...] * kmax)
    alpha = jnp.exp2(m_prev - m_next)
    p_e = jnp.exp2(s_e - m_next[:, :1]).astype(jnp.bfloat16)
    p_o = jnp.exp2(s_o - m_next[:, :1]).astype(jnp.bfloat16)
    vp_b = v_ref[...].astype(jnp.bfloat16)
    ones = jnp.ones((_BMP, 128 - _VALUE_DIM), dtype=jnp.bfloat16)
    va_e = jnp.concatenate([vp_b[:, :_VALUE_DIM], ones], axis=1)
    va_o = jnp.concatenate([vp_b[:, _VALUE_DIM:], ones], axis=1)
    pv = (jax.lax.dot_general(p_e, va_e, (((1,), (0,)), ((), ())),
                              preferred_element_type=jnp.float32)
          + jax.lax.dot_general(p_o, va_o, (((1,), (0,)), ((), ())),
                                preferred_element_type=jnp.float32))
    acc_scratch[...] = acc_scratch[...] * alpha[:, :1] + pv
    m_scratch[...] = m_next

    @pl.when(i == _NUM_M_BLOCKS - 1)
    def _fin():
        read = (acc_scratch[:, :_VALUE_DIM]
                / acc_scratch[:, _VALUE_DIM:_VALUE_DIM + 1])
        out = jax.lax.dot_general(read, wv_ref[...], (((1,), (0,)), ((), ())),
                                  preferred_element_type=jnp.float32)
        o_ref[...] = out + bv_ref[...]


def _attention(q2d, kp, vp, Wq, bq2, Wv, bv2, interpret=False):
    return pl.pallas_call(
        _attn_kernel,
        grid=(_NUM_M_BLOCKS,),
        in_specs=[
            pl.BlockSpec((_BQ, _KEY_DIM), lambda i: (0, 0)),
            pl.BlockSpec((_BMP, 128), lambda i: (i, 0)),
            pl.BlockSpec((_BMP, 128), lambda i: (i, 0)),
            pl.BlockSpec((_KEY_DIM, _KEY_DIM), lambda i: (0, 0)),
            pl.BlockSpec((1, _KEY_DIM), lambda i: (0, 0)),
            pl.BlockSpec((_VALUE_DIM, _VALUE_DIM), lambda i: (0, 0)),
            pl.BlockSpec((1, _VALUE_DIM), lambda i: (0, 0)),
        ],
        out_specs=pl.BlockSpec((_BQ, _VALUE_DIM), lambda i: (0, 0)),
        out_shape=jax.ShapeDtypeStruct((_BQ, _VALUE_DIM), jnp.float32),
        scratch_shapes=[
            pltpu.VMEM((_BQ, _KEY_DIM), jnp.bfloat16),
            pltpu.VMEM((_BQ, 128), jnp.float32),
            pltpu.VMEM((_BQ, 128), jnp.float32),
            pltpu.VMEM((_BQ, 128), jnp.float32),
        ],
        compiler_params=pltpu.CompilerParams(
            dimension_semantics=("arbitrary",)),
        interpret=interpret,
    )(q2d, kp, vp, Wq, bq2, Wv, bv2)


def kernel(queries, mem_keys, mem_values, Wq, bq, Wv, bv):
    b, n, _ = queries.shape
    q2d = queries.reshape(b * n, _KEY_DIM)
    kp = mem_keys.reshape(_MEMORY_SIZE // 2, 2 * _KEY_DIM)
    vp = mem_values.reshape(_MEMORY_SIZE // 2, 2 * _VALUE_DIM)
    out = _attention(q2d, kp, vp,
                     Wq, bq.reshape(1, -1), Wv, bv.reshape(1, -1))
    return out.reshape(b, n, _VALUE_DIM)


# speedup vs baseline: 2.0260x; 2.0260x over previous
import jax
import jax.numpy as jnp
from jax.experimental import pallas as pl
from jax.experimental.pallas import tpu as pltpu

_MEMORY_SIZE = 65536
_KEY_DIM = 64
_VALUE_DIM = 64
_BQ = 1024
_BM = 2048
_NUM_M_BLOCKS = _MEMORY_SIZE // _BM
_SCALE = 1.4426950408889634 / (_KEY_DIM ** 0.5)


def _attn_kernel(q_ref, kt_ref, vt_ref, wq_ref, bq_ref, wv_ref, bv_ref,
                 o_ref, q_scratch, qn_scratch, acc_scratch, m_scratch):
    i = pl.program_id(0)

    @pl.when(i == 0)
    def _init():
        q = jax.lax.dot_general(q_ref[...], wq_ref[...],
                                (((1,), (0,)), ((), ())),
                                preferred_element_type=jnp.float32)
        q_b = ((q + bq_ref[...]) * _SCALE).astype(jnp.bfloat16)
        q_scratch[...] = q_b
        q32 = q_b.astype(jnp.float32)
        qn = jnp.sqrt(jnp.sum(q32 * q32, axis=1, keepdims=True))
        qn_scratch[...] = jnp.broadcast_to(qn, qn_scratch.shape)
        m_scratch[...] = jnp.full_like(m_scratch, -jnp.inf)
        acc_scratch[...] = jnp.zeros_like(acc_scratch)

    kt = kt_ref[...]
    s = jax.lax.dot_general(q_scratch[...], kt.astype(jnp.bfloat16),
                            (((1,), (0,)), ((), ())),
                            preferred_element_type=jnp.float32)
    ksq = jnp.sum(kt * kt, axis=0, keepdims=True)
    kmax = jnp.sqrt(jnp.max(ksq)) * 1.01
    m_prev = m_scratch[...]
    m_next = jnp.maximum(m_prev, qn_scratch[...] * kmax)
    alpha = jnp.exp2(m_prev - m_next)
    p_b = jnp.exp2(s - m_next[:, :1]).astype(jnp.bfloat16)
    vt_aug = jnp.concatenate(
        [vt_ref[...].astype(jnp.bfloat16),
         jnp.ones((128 - _VALUE_DIM, _BM), dtype=jnp.bfloat16)], axis=0)
    pv = jax.lax.dot_general(p_b, vt_aug, (((1,), (1,)), ((), ())),
                             preferred_element_type=jnp.float32)
    acc_scratch[...] = acc_scratch[...] * alpha[:, :1] + pv
    m_scratch[...] = m_next

    @pl.when(i == _NUM_M_BLOCKS - 1)
    def _fin():
        read = (acc_scratch[:, :_VALUE_DIM]
                / acc_scratch[:, _VALUE_DIM:_VALUE_DIM + 1])
        out = jax.lax.dot_general(read, wv_ref[...], (((1,), (0,)), ((), ())),
                                  preferred_element_type=jnp.float32)
        o_ref[...] = out + bv_ref[...]


def _attention(q2d, kt, vt, Wq, bq2, Wv, bv2, interpret=False):
    return pl.pallas_call(
        _attn_kernel,
        grid=(_NUM_M_BLOCKS,),
        in_specs=[
            pl.BlockSpec((_BQ, _KEY_DIM), lambda i: (0, 0)),
            pl.BlockSpec((_KEY_DIM, _BM), lambda i: (0, i)),
            pl.BlockSpec((_VALUE_DIM, _BM), lambda i: (0, i)),
            pl.BlockSpec((_KEY_DIM, _KEY_DIM), lambda i: (0, 0)),
            pl.BlockSpec((1, _KEY_DIM), lambda i: (0, 0)),
            pl.BlockSpec((_VALUE_DIM, _VALUE_DIM), lambda i: (0, 0)),
            pl.BlockSpec((1, _VALUE_DIM), lambda i: (0, 0)),
        ],
        out_specs=pl.BlockSpec((_BQ, _VALUE_DIM), lambda i: (0, 0)),
        out_shape=jax.ShapeDtypeStruct((_BQ, _VALUE_DIM), jnp.float32),
        scratch_shapes=[
            pltpu.VMEM((_BQ, _KEY_DIM), jnp.bfloat16),
            pltpu.VMEM((_BQ, 128), jnp.float32),
            pltpu.VMEM((_BQ, 128), jnp.float32),
            pltpu.VMEM((_BQ, 128), jnp.float32),
        ],
        compiler_params=pltpu.CompilerParams(
            dimension_semantics=("arbitrary",)),
        interpret=interpret,
    )(q2d, kt, vt, Wq, bq2, Wv, bv2)


def kernel(queries, mem_keys, mem_values, Wq, bq, Wv, bv):
    b, n, _ = queries.shape
    q2d = queries.reshape(b * n, _KEY_DIM)
    out = _attention(q2d, mem_keys.T, mem_values.T,
                     Wq, bq.reshape(1, -1), Wv, bv.reshape(1, -1))
    return out.reshape(b, n, _VALUE_DIM)


# speedup vs baseline: 2.1417x; 1.0571x over previous
import jax
import jax.numpy as jnp
from jax.experimental import pallas as pl
from jax.experimental.pallas import tpu as pltpu

_MEMORY_SIZE = 65536
_KEY_DIM = 64
_VALUE_DIM = 64
_BQ = 1024
_BM = 4096
_NUM_M_BLOCKS = _MEMORY_SIZE // _BM
_SCALE = 1.4426950408889634 / (_KEY_DIM ** 0.5)


def _attn_kernel(q_ref, kt_ref, vt_ref, wq_ref, bq_ref, wv_ref, bv_ref,
                 o_ref, q_scratch, qn_scratch, acc_scratch, m_scratch):
    i = pl.program_id(0)

    @pl.when(i == 0)
    def _init():
        q = jax.lax.dot_general(q_ref[...], wq_ref[...],
                                (((1,), (0,)), ((), ())),
                                preferred_element_type=jnp.float32)
        q_b = ((q + bq_ref[...]) * _SCALE).astype(jnp.bfloat16)
        q_scratch[...] = q_b
        q32 = q_b.astype(jnp.float32)
        qn = jnp.sqrt(jnp.sum(q32 * q32, axis=1, keepdims=True))
        qn_scratch[...] = jnp.broadcast_to(qn, qn_scratch.shape)
        m_scratch[...] = jnp.full_like(m_scratch, -jnp.inf)
        acc_scratch[...] = jnp.zeros_like(acc_scratch)

    kt = kt_ref[...]
    s = jax.lax.dot_general(q_scratch[...], kt.astype(jnp.bfloat16),
                            (((1,), (0,)), ((), ())),
                            preferred_element_type=jnp.float32)
    ksq = jnp.sum(kt * kt, axis=0, keepdims=True)
    kmax = jnp.sqrt(jnp.max(ksq)) * 1.01
    m_prev = m_scratch[...]
    m_next = jnp.maximum(m_prev, qn_scratch[...] * kmax)
    alpha = jnp.exp2(m_prev - m_next)
    p_b = jnp.exp2(s - m_next[:, :1]).astype(jnp.bfloat16)
    vt_aug = jnp.concatenate(
        [vt_ref[...].astype(jnp.bfloat16),
         jnp.ones((128 - _VALUE_DIM, _BM), dtype=jnp.bfloat16)], axis=0)
    pv = jax.lax.dot_general(p_b, vt_aug, (((1,), (1,)), ((), ())),
                             preferred_element_type=jnp.float32)
    acc_scratch[...] = acc_scratch[...] * alpha[:, :1] + pv
    m_scratch[...] = m_next

    @pl.when(i == _NUM_M_BLOCKS - 1)
    def _fin():
        read = (acc_scratch[:, :_VALUE_DIM]
                / acc_scratch[:, _VALUE_DIM:_VALUE_DIM + 1])
        out = jax.lax.dot_general(read, wv_ref[...], (((1,), (0,)), ((), ())),
                                  preferred_element_type=jnp.float32)
        o_ref[...] = out + bv_ref[...]


def _attention(q2d, kt, vt, Wq, bq2, Wv, bv2, interpret=False):
    return pl.pallas_call(
        _attn_kernel,
        grid=(_NUM_M_BLOCKS,),
        in_specs=[
            pl.BlockSpec((_BQ, _KEY_DIM), lambda i: (0, 0)),
            pl.BlockSpec((_KEY_DIM, _BM), lambda i: (0, i)),
            pl.BlockSpec((_VALUE_DIM, _BM), lambda i: (0, i)),
            pl.BlockSpec((_KEY_DIM, _KEY_DIM), lambda i: (0, 0)),
            pl.BlockSpec((1, _KEY_DIM), lambda i: (0, 0)),
            pl.BlockSpec((_VALUE_DIM, _VALUE_DIM), lambda i: (0, 0)),
            pl.BlockSpec((1, _VALUE_DIM), lambda i: (0, 0)),
        ],
        out_specs=pl.BlockSpec((_BQ, _VALUE_DIM), lambda i: (0, 0)),
        out_shape=jax.ShapeDtypeStruct((_BQ, _VALUE_DIM), jnp.float32),
        scratch_shapes=[
            pltpu.VMEM((_BQ, _KEY_DIM), jnp.bfloat16),
            pltpu.VMEM((_BQ, 128), jnp.float32),
            pltpu.VMEM((_BQ, 128), jnp.float32),
            pltpu.VMEM((_BQ, 128), jnp.float32),
        ],
        compiler_params=pltpu.CompilerParams(
            dimension_semantics=("arbitrary",)),
        interpret=interpret,
    )(q2d, kt, vt, Wq, bq2, Wv, bv2)


def kernel(queries, mem_keys, mem_values, Wq, bq, Wv, bv):
    b, n, _ = queries.shape
    q2d = queries.reshape(b * n, _KEY_DIM)
    out = _attention(q2d, mem_keys.T, mem_values.T,
                     Wq, bq.reshape(1, -1), Wv, bv.reshape(1, -1))
    return out.reshape(b, n, _VALUE_DIM)


# speedup vs baseline: 2.1800x; 1.0179x over previous
import jax
import jax.numpy as jnp
from jax.experimental import pallas as pl
from jax.experimental.pallas import tpu as pltpu

_MEMORY_SIZE = 65536
_KEY_DIM = 64
_VALUE_DIM = 64
_BQ = 1024
_BM = 8192
_NUM_M_BLOCKS = _MEMORY_SIZE // _BM
_SCALE = 1.4426950408889634 / (_KEY_DIM ** 0.5)


def _attn_kernel(q_ref, kt_ref, vt_ref, wq_ref, bq_ref, wv_ref, bv_ref,
                 o_ref, q_scratch, qn_scratch, acc_scratch, m_scratch):
    i = pl.program_id(0)

    @pl.when(i == 0)
    def _init():
        q = jax.lax.dot_general(q_ref[...], wq_ref[...],
                                (((1,), (0,)), ((), ())),
                                preferred_element_type=jnp.float32)
        q_b = ((q + bq_ref[...]) * _SCALE).astype(jnp.bfloat16)
        q_scratch[...] = q_b
        q32 = q_b.astype(jnp.float32)
        qn = jnp.sqrt(jnp.sum(q32 * q32, axis=1, keepdims=True))
        qn_scratch[...] = jnp.broadcast_to(qn, qn_scratch.shape)
        m_scratch[...] = jnp.full_like(m_scratch, -jnp.inf)
        acc_scratch[...] = jnp.zeros_like(acc_scratch)

    kt = kt_ref[...]
    s = jax.lax.dot_general(q_scratch[...], kt.astype(jnp.bfloat16),
                            (((1,), (0,)), ((), ())),
                            preferred_element_type=jnp.float32)
    ksq = jnp.sum(kt * kt, axis=0, keepdims=True)
    kmax = jnp.sqrt(jnp.max(ksq)) * 1.01
    m_prev = m_scratch[...]
    m_next = jnp.maximum(m_prev, qn_scratch[...] * kmax)
    alpha = jnp.exp2(m_prev - m_next)
    p_b = jnp.exp2(s - m_next[:, :1]).astype(jnp.bfloat16)
    vt_aug = jnp.concatenate(
        [vt_ref[...].astype(jnp.bfloat16),
         jnp.ones((128 - _VALUE_DIM, _BM), dtype=jnp.bfloat16)], axis=0)
    pv = jax.lax.dot_general(p_b, vt_aug, (((1,), (1,)), ((), ())),
                             preferred_element_type=jnp.float32)
    acc_scratch[...] = acc_scratch[...] * alpha[:, :1] + pv
    m_scratch[...] = m_next

    @pl.when(i == _NUM_M_BLOCKS - 1)
    def _fin():
        read = (acc_scratch[:, :_VALUE_DIM]
                / acc_scratch[:, _VALUE_DIM:_VALUE_DIM + 1])
        out = jax.lax.dot_general(read, wv_ref[...], (((1,), (0,)), ((), ())),
                                  preferred_element_type=jnp.float32)
        o_ref[...] = out + bv_ref[...]


def _attention(q2d, kt, vt, Wq, bq2, Wv, bv2, interpret=False):
    return pl.pallas_call(
        _attn_kernel,
        grid=(_NUM_M_BLOCKS,),
        in_specs=[
            pl.BlockSpec((_BQ, _KEY_DIM), lambda i: (0, 0)),
            pl.BlockSpec((_KEY_DIM, _BM), lambda i: (0, i)),
            pl.BlockSpec((_VALUE_DIM, _BM), lambda i: (0, i)),
            pl.BlockSpec((_KEY_DIM, _KEY_DIM), lambda i: (0, 0)),
            pl.BlockSpec((1, _KEY_DIM), lambda i: (0, 0)),
            pl.BlockSpec((_VALUE_DIM, _VALUE_DIM), lambda i: (0, 0)),
            pl.BlockSpec((1, _VALUE_DIM), lambda i: (0, 0)),
        ],
        out_specs=pl.BlockSpec((_BQ, _VALUE_DIM), lambda i: (0, 0)),
        out_shape=jax.ShapeDtypeStruct((_BQ, _VALUE_DIM), jnp.float32),
        scratch_shapes=[
            pltpu.VMEM((_BQ, _KEY_DIM), jnp.bfloat16),
            pltpu.VMEM((_BQ, 128), jnp.float32),
            pltpu.VMEM((_BQ, 128), jnp.float32),
            pltpu.VMEM((_BQ, 128), jnp.float32),
        ],
        compiler_params=pltpu.CompilerParams(
            dimension_semantics=("arbitrary",)),
        interpret=interpret,
    )(q2d, kt, vt, Wq, bq2, Wv, bv2)


def kernel(queries, mem_keys, mem_values, Wq, bq, Wv, bv):
    b, n, _ = queries.shape
    q2d = queries.reshape(b * n, _KEY_DIM)
    out = _attention(q2d, mem_keys.T, mem_values.T,
                     Wq, bq.reshape(1, -1), Wv, bv.reshape(1, -1))
    return out.reshape(b, n, _VALUE_DIM)


# speedup vs baseline: 2.2946x; 1.0526x over previous
import jax
import jax.numpy as jnp
from jax.experimental import pallas as pl
from jax.experimental.pallas import tpu as pltpu

_MEMORY_SIZE = 65536
_KEY_DIM = 64
_VALUE_DIM = 64
_BQ = 1024
_BATCH = 8
_BM = 8192
_NUM_M_BLOCKS = _MEMORY_SIZE // _BM
_SCALE = 1.4426950408889634 / (_KEY_DIM ** 0.5)


def _attn_kernel(qt_ref, kt_ref, vt_ref, wq_ref, bq_ref, wv_ref, bv_ref,
                 o_ref, q_scratch, qn_scratch, acc_scratch, m_scratch):
    i = pl.program_id(0)

    @pl.when(i == 0)
    def _init():
        q_raw = jnp.concatenate(
            [jnp.transpose(qt_ref[b * 64:(b + 1) * 64, :], (1, 0))
             for b in range(_BATCH)], axis=0)
        q = jax.lax.dot_general(q_raw, wq_ref[...],
                                (((1,), (0,)), ((), ())),
                                preferred_element_type=jnp.float32)
        q_b = ((q + bq_ref[...]) * _SCALE).astype(jnp.bfloat16)
        q_scratch[...] = q_b
        q32 = q_b.astype(jnp.float32)
        qn = jnp.sqrt(jnp.sum(q32 * q32, axis=1, keepdims=True))
        qn_scratch[...] = jnp.broadcast_to(qn, qn_scratch.shape)
        m_scratch[...] = jnp.full_like(m_scratch, -jnp.inf)
        acc_scratch[...] = jnp.zeros_like(acc_scratch)

    kt = kt_ref[...]
    s = jax.lax.dot_general(q_scratch[...], kt.astype(jnp.bfloat16),
                            (((1,), (0,)), ((), ())),
                            preferred_element_type=jnp.float32)
    ksq = jnp.sum(kt * kt, axis=0, keepdims=True)
    kmax = jnp.sqrt(jnp.max(ksq)) * 1.01
    m_prev = m_scratch[...]
    m_next = jnp.maximum(m_prev, qn_scratch[...] * kmax)
    alpha = jnp.exp2(m_prev - m_next)
    p_b = jnp.exp2(s - m_next[:, :1]).astype(jnp.bfloat16)
    vt_aug = jnp.concatenate(
        [vt_ref[...].astype(jnp.bfloat16),
         jnp.ones((128 - _VALUE_DIM, _BM), dtype=jnp.bfloat16)], axis=0)
    pv = jax.lax.dot_general(p_b, vt_aug, (((1,), (1,)), ((), ())),
                             preferred_element_type=jnp.float32)
    acc_scratch[...] = acc_scratch[...] * alpha[:, :1] + pv
    m_scratch[...] = m_next

    @pl.when(i == _NUM_M_BLOCKS - 1)
    def _fin():
        read = (acc_scratch[:, :_VALUE_DIM]
                / acc_scratch[:, _VALUE_DIM:_VALUE_DIM + 1])
        out = jax.lax.dot_general(read, wv_ref[...], (((1,), (0,)), ((), ())),
                                  preferred_element_type=jnp.float32)
        out = out + bv_ref[...]
        for b in range(_BATCH):
            o_ref[b * 64:(b + 1) * 64, :] = jnp.transpose(
                out[b * 128:(b + 1) * 128, :], (1, 0))


def _attention(qt, kt, vt, Wq, bq2, Wv, bv2, interpret=False):
    return pl.pallas_call(
        _attn_kernel,
        grid=(_NUM_M_BLOCKS,),
        in_specs=[
            pl.BlockSpec((_BATCH * _KEY_DIM, 128), lambda i: (0, 0)),
            pl.BlockSpec((_KEY_DIM, _BM), lambda i: (0, i)),
            pl.BlockSpec((_VALUE_DIM, _BM), lambda i: (0, i)),
            pl.BlockSpec((_KEY_DIM, _KEY_DIM), lambda i: (0, 0)),
            pl.BlockSpec((1, _KEY_DIM), lambda i: (0, 0)),
            pl.BlockSpec((_VALUE_DIM, _VALUE_DIM), lambda i: (0, 0)),
            pl.BlockSpec((1, _VALUE_DIM), lambda i: (0, 0)),
        ],
        out_specs=pl.BlockSpec((_BATCH * _VALUE_DIM, 128), lambda i: (0, 0)),
        out_shape=jax.ShapeDtypeStruct((_BATCH * _VALUE_DIM, 128),
                                       jnp.float32),
        scratch_shapes=[
            pltpu.VMEM((_BQ, _KEY_DIM), jnp.bfloat16),
            pltpu.VMEM((_BQ, 128), jnp.float32),
            pltpu.VMEM((_BQ, 128), jnp.float32),
            pltpu.VMEM((_BQ, 128), jnp.float32),
        ],
        compiler_params=pltpu.CompilerParams(
            dimension_semantics=("arbitrary",)),
        interpret=interpret,
    )(qt, kt, vt, Wq, bq2, Wv, bv2)


def kernel(queries, mem_keys, mem_values, Wq, bq, Wv, bv):
    b, n, _ = queries.shape
    qt = queries.transpose(0, 2, 1).reshape(b * _KEY_DIM, n)
    out = _attention(qt, mem_keys.T, mem_values.T,
                     Wq, bq.reshape(1, -1), Wv, bv.reshape(1, -1))
    return out.reshape(b, _VALUE_DIM, n).transpose(0, 2, 1)
